# Initial kernel scaffold; baseline (speedup 1.0000x reference)
import jax
import jax.numpy as jnp
from jax.experimental import pallas as pl


def kernel(x, edge_index, W_l, b_l, W_r, W1, W2):
    raise NotImplementedError("write your pallas kernel here")


# speedup vs baseline: 8.1019x; 8.1019x over previous
import functools

import jax
import jax.numpy as jnp
from jax import lax
from jax.experimental import pallas as pl
from jax.experimental.pallas import tpu as pltpu
from jax.experimental.pallas import tpu_sc as plsc

_N = 10000
_E = 320000
_D = 128

_NC = 2
_NS = 16
_NW = _NC * _NS
_K = 128
_CPW = -(-_E // (_NW * _K))
_EPAD = _NW * _CPW * _K
_ROWS_PT = 632
_NSH = _NS * _ROWS_PT
_CNT_PT = 1024
_NCNT = _NS * _CNT_PT


def _sc_aggregate_body(x_hbm, src_hbm, dst_hbm, z2_hbm, zc_hbm,
                       sums_hbm, cnts_hbm,
                       idx_s_v, idx_d_v, rows_v, ones_v,
                       shared_sum, shared_cnt, sem):
    c = lax.axis_index("c")
    s = lax.axis_index("s")
    wid = c * _NS + s

    pltpu.sync_copy(src_hbm.at[wid], idx_s_v)
    pltpu.sync_copy(dst_hbm.at[wid], idx_d_v)

    pltpu.sync_copy(z2_hbm, shared_sum.at[pl.ds(s * _ROWS_PT, _ROWS_PT), :])
    pltpu.sync_copy(zc_hbm, shared_cnt.at[pl.ds(s * _CNT_PT, _CNT_PT)])
    for i in range(_K // 16):
        ones_v[pl.ds(i * 16, 16)] = jnp.ones((16,), jnp.float32)
    plsc.subcore_barrier()

    def step(j, carry):
        pltpu.async_copy(x_hbm.at[idx_s_v.at[j]], rows_v, sem).wait()
        pltpu.sync_copy(rows_v, shared_sum.at[idx_d_v.at[j]], add=True)
        pltpu.sync_copy(ones_v, shared_cnt.at[idx_d_v.at[j]], add=True)
        return carry

    lax.fori_loop(0, _CPW, step, 0)
    plsc.subcore_barrier()

    row0 = s * _ROWS_PT
    pltpu.sync_copy(shared_sum.at[pl.ds(row0, _ROWS_PT), :],
                    sums_hbm.at[c, pl.ds(row0, _ROWS_PT), :])
    el0 = s * _CNT_PT
    pltpu.sync_copy(shared_cnt.at[pl.ds(el0, _CNT_PT)],
                    cnts_hbm.at[pl.ds(c * _NCNT + el0, _CNT_PT)])


@functools.cache
def _get_sc_aggregate():
    return pl.kernel(
        _sc_aggregate_body,
        out_type=(jax.ShapeDtypeStruct((_NC, _NSH, _D), jnp.float32),
                  jax.ShapeDtypeStruct((_NC * _NCNT,), jnp.float32)),
        mesh=plsc.VectorSubcoreMesh(core_axis_name="c", subcore_axis_name="s"),
        scratch_types=(
            pltpu.VMEM((_CPW, _K), jnp.int32),
            pltpu.VMEM((_CPW, _K), jnp.int32),
            pltpu.VMEM((_K, _D), jnp.float32),
            pltpu.VMEM((_K,), jnp.float32),
            pltpu.VMEM_SHARED((_NSH, _D), jnp.float32),
            pltpu.VMEM_SHARED((_NCNT,), jnp.float32),
            pltpu.SemaphoreType.DMA,
        ),
    )


def _tc_dense_body(x_ref, sums_ref, cnts_ref, wl_ref, bl_ref, wr_ref, w12_ref,
                   x1_ref, out_ref):
    ssum = sums_ref[0] + sums_ref[1]
    cnt = cnts_ref[0] + cnts_ref[1]
    mean = ssum / jnp.maximum(cnt, 1.0)
    x1 = (jnp.dot(mean, wl_ref[...], preferred_element_type=jnp.float32)
          + bl_ref[...]
          + jnp.dot(x_ref[...], wr_ref[...], preferred_element_type=jnp.float32))
    x1_ref[...] = x1
    rn = jnp.sqrt(jnp.sum(x1 * x1, axis=1, keepdims=True))
    hn = x1 / jnp.maximum(rn, 1e-12)
    w = w12_ref[...]
    wn = jnp.sqrt(jnp.sum(w * w, axis=0, keepdims=True))
    out_ref[...] = jnp.dot(hn, w / jnp.maximum(wn, 1e-12),
                           preferred_element_type=jnp.float32)


_BR = 1000


def _tc_dense(x, sums, cnts3, w_l, b_l2, w_r, w12):
    grid = (_N // _BR,)
    return pl.pallas_call(
        _tc_dense_body,
        grid=grid,
        in_specs=[
            pl.BlockSpec((_BR, _D), lambda i: (i, 0)),
            pl.BlockSpec((_NC, _BR, _D), lambda i: (0, i, 0)),
            pl.BlockSpec((_NC, _BR, 1), lambda i: (0, i, 0)),
            pl.BlockSpec((_D, _D), lambda i: (0, 0)),
            pl.BlockSpec((1, _D), lambda i: (0, 0)),
            pl.BlockSpec((_D, _D), lambda i: (0, 0)),
            pl.BlockSpec((_D, 256), lambda i: (0, 0)),
        ],
        out_specs=[
            pl.BlockSpec((_BR, _D), lambda i: (i, 0)),
            pl.BlockSpec((_BR, 256), lambda i: (i, 0)),
        ],
        out_shape=[
            jax.ShapeDtypeStruct((_N, _D), jnp.float32),
            jax.ShapeDtypeStruct((_N, 256), jnp.float32),
        ],
    )(x, sums, cnts3, w_l, b_l2, w_r, w12)


def kernel(x, edge_index, W_l, b_l, W_r, W1, W2):
    src = edge_index[0]
    dst = edge_index[1]
    npad = _EPAD - _E
    pad_src = (jnp.arange(npad, dtype=jnp.int32) * 97) % _N
    pad_dst = _N + (jnp.arange(npad, dtype=jnp.int32) % (_NSH - _N))
    src3 = jnp.concatenate([src, pad_src]).reshape(_NW, _CPW, _K)
    dst3 = jnp.concatenate([dst, pad_dst]).reshape(_NW, _CPW, _K)
    z2 = jnp.zeros((_ROWS_PT, _D), jnp.float32)
    zc = jnp.zeros((_CNT_PT,), jnp.float32)

    sums, cnts = _get_sc_aggregate()(x, src3, dst3, z2, zc)

    w12 = jnp.zeros((_D, 256), jnp.float32)
    w12 = lax.dynamic_update_slice(w12, W1, (0, 0))
    w12 = lax.dynamic_update_slice(w12, W2, (0, W1.shape[1]))
    cnts3 = jnp.stack([cnts[:_N], cnts[_NCNT:_NCNT + _N]])[:, :, None]
    x1, out12 = _tc_dense(x, sums, cnts3, W_l, b_l.reshape(1, _D), W_r, w12)

    c1 = W1.shape[1]
    c2 = W2.shape[1]
    return (out12[:, :c1], out12[:, c1:c1 + c2], x1)


# speedup vs baseline: 10.4881x; 1.2945x over previous
import functools

import jax
import jax.numpy as jnp
from jax import lax
from jax.experimental import pallas as pl
from jax.experimental.pallas import tpu as pltpu
from jax.experimental.pallas import tpu_sc as plsc

_N = 10000
_E = 320000
_D = 128

_NC = 2
_NS = 16
_NW = _NC * _NS
_K = 128
_CPW = 80
_IBLK = 16
_EPAD = _NW * _CPW * _K
_ROWS_PT = 632
_NSH = _NS * _ROWS_PT
_CNT_PT = 1024
_NCNT = _NS * _CNT_PT


def _sc_aggregate_body(x_hbm, src_hbm, dst_hbm, z2_hbm, zc_hbm,
                       sums_hbm, cnts_hbm,
                       idx_s_v, idx_d_v, rows_v, ones_v,
                       shared_sum, shared_cnt, sem_a, sem_b):
    c = lax.axis_index("c")
    s = lax.axis_index("s")
    wid = c * _NS + s

    pltpu.sync_copy(z2_hbm, shared_sum.at[pl.ds(s * _ROWS_PT, _ROWS_PT), :])
    pltpu.sync_copy(zc_hbm, shared_cnt.at[pl.ds(s * _CNT_PT, _CNT_PT)])
    for i in range(_K // 16):
        ones_v[pl.ds(i * 16, 16)] = jnp.ones((16,), jnp.float32)
    plsc.subcore_barrier()

    def _wait_gather(buf, sem):
        pltpu.make_async_copy(x_hbm.at[pl.ds(0, _K), :], rows_v.at[buf],
                              sem).wait()

    def _consume(j, buf):
        pltpu.sync_copy(rows_v.at[buf], shared_sum.at[idx_d_v.at[j]], add=True)
        pltpu.sync_copy(ones_v, shared_cnt.at[idx_d_v.at[j]], add=True)

    def pair(g, carry):
        j0 = 2 * g
        j1 = j0 + 1
        pltpu.async_copy(x_hbm.at[idx_s_v.at[j1]], rows_v.at[1], sem_b)
        _wait_gather(0, sem_a)
        _consume(j0, 0)
        nxt = jnp.minimum(j1 + 1, _IBLK - 1)
        pltpu.async_copy(x_hbm.at[idx_s_v.at[nxt]], rows_v.at[0], sem_a)
        _wait_gather(1, sem_b)
        _consume(j1, 1)
        return carry

    for blk in range(_CPW // _IBLK):
        pltpu.sync_copy(src_hbm.at[wid, pl.ds(blk * _IBLK, _IBLK)], idx_s_v)
        pltpu.sync_copy(dst_hbm.at[wid, pl.ds(blk * _IBLK, _IBLK)], idx_d_v)
        pltpu.async_copy(x_hbm.at[idx_s_v.at[0]], rows_v.at[0], sem_a)
        lax.fori_loop(0, _IBLK // 2, pair, 0)
        _wait_gather(0, sem_a)
    plsc.subcore_barrier()

    row0 = s * _ROWS_PT
    pltpu.sync_copy(shared_sum.at[pl.ds(row0, _ROWS_PT), :],
                    sums_hbm.at[c, pl.ds(row0, _ROWS_PT), :])
    el0 = s * _CNT_PT
    pltpu.sync_copy(shared_cnt.at[pl.ds(el0, _CNT_PT)],
                    cnts_hbm.at[pl.ds(c * _NCNT + el0, _CNT_PT)])


@functools.cache
def _get_sc_aggregate():
    return pl.kernel(
        _sc_aggregate_body,
        out_type=(jax.ShapeDtypeStruct((_NC, _NSH, _D), jnp.float32),
                  jax.ShapeDtypeStruct((_NC * _NCNT,), jnp.float32)),
        mesh=plsc.VectorSubcoreMesh(core_axis_name="c", subcore_axis_name="s"),
        scratch_types=(
            pltpu.VMEM((_IBLK, _K), jnp.int32),
            pltpu.VMEM((_IBLK, _K), jnp.int32),
            pltpu.VMEM((2, _K, _D), jnp.float32),
            pltpu.VMEM((_K,), jnp.float32),
            pltpu.VMEM_SHARED((_NSH, _D), jnp.float32),
            pltpu.VMEM_SHARED((_NCNT,), jnp.float32),
            pltpu.SemaphoreType.DMA,
            pltpu.SemaphoreType.DMA,
        ),
    )


def _tc_dense_body(x_ref, sums_ref, cnts_ref, wl_ref, bl_ref, wr_ref, w12_ref,
                   x1_ref, out_ref):
    ssum = sums_ref[0] + sums_ref[1]
    cnt = cnts_ref[0] + cnts_ref[1]
    mean = ssum / jnp.maximum(cnt, 1.0)
    x1 = (jnp.dot(mean, wl_ref[...], preferred_element_type=jnp.float32)
          + bl_ref[...]
          + jnp.dot(x_ref[...], wr_ref[...], preferred_element_type=jnp.float32))
    x1_ref[...] = x1
    rn = jnp.sqrt(jnp.sum(x1 * x1, axis=1, keepdims=True))
    hn = x1 / jnp.maximum(rn, 1e-12)
    w = w12_ref[...]
    wn = jnp.sqrt(jnp.sum(w * w, axis=0, keepdims=True))
    out_ref[...] = jnp.dot(hn, w / jnp.maximum(wn, 1e-12),
                           preferred_element_type=jnp.float32)


_BR = 1000


def _tc_dense(x, sums, cnts3, w_l, b_l2, w_r, w12):
    grid = (_N // _BR,)
    return pl.pallas_call(
        _tc_dense_body,
        grid=grid,
        in_specs=[
            pl.BlockSpec((_BR, _D), lambda i: (i, 0)),
            pl.BlockSpec((_NC, _BR, _D), lambda i: (0, i, 0)),
            pl.BlockSpec((_NC, _BR, 1), lambda i: (0, i, 0)),
            pl.BlockSpec((_D, _D), lambda i: (0, 0)),
            pl.BlockSpec((1, _D), lambda i: (0, 0)),
            pl.BlockSpec((_D, _D), lambda i: (0, 0)),
            pl.BlockSpec((_D, 256), lambda i: (0, 0)),
        ],
        out_specs=[
            pl.BlockSpec((_BR, _D), lambda i: (i, 0)),
            pl.BlockSpec((_BR, 256), lambda i: (i, 0)),
        ],
        out_shape=[
            jax.ShapeDtypeStruct((_N, _D), jnp.float32),
            jax.ShapeDtypeStruct((_N, 256), jnp.float32),
        ],
    )(x, sums, cnts3, w_l, b_l2, w_r, w12)


def kernel(x, edge_index, W_l, b_l, W_r, W1, W2):
    src = edge_index[0]
    dst = edge_index[1]
    npad = _EPAD - _E
    pad_src = (jnp.arange(npad, dtype=jnp.int32) * 97) % _N
    pad_dst = _N + (jnp.arange(npad, dtype=jnp.int32) % (_NSH - _N))
    src3 = jnp.concatenate([src, pad_src]).reshape(_NW, _CPW, _K)
    dst3 = jnp.concatenate([dst, pad_dst]).reshape(_NW, _CPW, _K)
    z2 = jnp.zeros((_ROWS_PT, _D), jnp.float32)
    zc = jnp.zeros((_CNT_PT,), jnp.float32)

    sums, cnts = _get_sc_aggregate()(x, src3, dst3, z2, zc)

    w12 = jnp.zeros((_D, 256), jnp.float32)
    w12 = lax.dynamic_update_slice(w12, W1, (0, 0))
    w12 = lax.dynamic_update_slice(w12, W2, (0, W1.shape[1]))
    cnts3 = jnp.stack([cnts[:_N], cnts[_NCNT:_NCNT + _N]])[:, :, None]
    x1, out12 = _tc_dense(x, sums, cnts3, W_l, b_l.reshape(1, _D), W_r, w12)

    c1 = W1.shape[1]
    c2 = W2.shape[1]
    return (out12[:, :c1], out12[:, c1:c1 + c2], x1)


# speedup vs baseline: 11.2928x; 1.0767x over previous
import functools

import jax
import jax.numpy as jnp
from jax import lax
from jax.experimental import pallas as pl
from jax.experimental.pallas import tpu as pltpu
from jax.experimental.pallas import tpu_sc as plsc

_N = 10000
_E = 320000
_D = 128
_C1 = 50
_C2 = 100

_NC = 2
_NS = 16
_NW = _NC * _NS
_K = 128
_CPW = 80
_IBLK = 16
_EPAD = _NW * _CPW * _K
_ROWS_PT = 632
_NSH = _NS * _ROWS_PT
_CNT_PT = 1024
_NCNT = _NS * _CNT_PT


def _sc_aggregate_body(x_hbm, src_hbm, dst_hbm,
                       sums_hbm, cnts_hbm,
                       idx_s_v, idx_d_v, rows_v, ones_v, zc_v,
                       shared_sum, shared_cnt, sem_a, sem_b):
    c = lax.axis_index("c")
    s = lax.axis_index("s")
    wid = c * _NS + s

    def _z(i, carry):
        rows_v[1, i, pl.ds(0, 16)] = jnp.zeros((16,), jnp.float32)
        rows_v[1, i, pl.ds(16, 16)] = jnp.zeros((16,), jnp.float32)
        rows_v[1, i, pl.ds(32, 16)] = jnp.zeros((16,), jnp.float32)
        rows_v[1, i, pl.ds(48, 16)] = jnp.zeros((16,), jnp.float32)
        rows_v[1, i, pl.ds(64, 16)] = jnp.zeros((16,), jnp.float32)
        rows_v[1, i, pl.ds(80, 16)] = jnp.zeros((16,), jnp.float32)
        rows_v[1, i, pl.ds(96, 16)] = jnp.zeros((16,), jnp.float32)
        rows_v[1, i, pl.ds(112, 16)] = jnp.zeros((16,), jnp.float32)
        return carry

    lax.fori_loop(0, _K, _z, 0)
    for i in range(_CNT_PT // 16):
        zc_v[pl.ds(i * 16, 16)] = jnp.zeros((16,), jnp.float32)
    r0 = s * _ROWS_PT
    for k in range(4):
        pltpu.sync_copy(rows_v.at[1],
                        shared_sum.at[pl.ds(r0 + k * _K, _K), :])
    pltpu.sync_copy(rows_v.at[1, pl.ds(0, _ROWS_PT - 4 * _K), :],
                    shared_sum.at[pl.ds(r0 + 4 * _K, _ROWS_PT - 4 * _K), :])
    pltpu.sync_copy(zc_v, shared_cnt.at[pl.ds(s * _CNT_PT, _CNT_PT)])
    for i in range(_K // 16):
        ones_v[pl.ds(i * 16, 16)] = jnp.ones((16,), jnp.float32)
    plsc.subcore_barrier()

    def _wait_gather(buf, sem):
        pltpu.make_async_copy(x_hbm.at[pl.ds(0, _K), :], rows_v.at[buf],
                              sem).wait()

    def _consume(j, buf):
        pltpu.sync_copy(rows_v.at[buf], shared_sum.at[idx_d_v.at[j]], add=True)
        pltpu.sync_copy(ones_v, shared_cnt.at[idx_d_v.at[j]], add=True)

    def pair(g, carry):
        j0 = 2 * g
        j1 = j0 + 1
        pltpu.async_copy(x_hbm.at[idx_s_v.at[j1]], rows_v.at[1], sem_b)
        _wait_gather(0, sem_a)
        _consume(j0, 0)
        nxt = jnp.minimum(j1 + 1, _IBLK - 1)
        pltpu.async_copy(x_hbm.at[idx_s_v.at[nxt]], rows_v.at[0], sem_a)
        _wait_gather(1, sem_b)
        _consume(j1, 1)
        return carry

    for blk in range(_CPW // _IBLK):
        pltpu.sync_copy(src_hbm.at[wid, pl.ds(blk * _IBLK, _IBLK)], idx_s_v)
        pltpu.sync_copy(dst_hbm.at[wid, pl.ds(blk * _IBLK, _IBLK)], idx_d_v)
        pltpu.async_copy(x_hbm.at[idx_s_v.at[0]], rows_v.at[0], sem_a)
        lax.fori_loop(0, _IBLK // 2, pair, 0)
        _wait_gather(0, sem_a)
    plsc.subcore_barrier()

    row0 = s * _ROWS_PT
    pltpu.sync_copy(shared_sum.at[pl.ds(row0, _ROWS_PT), :],
                    sums_hbm.at[c, pl.ds(row0, _ROWS_PT), :])
    el0 = s * _CNT_PT
    pltpu.sync_copy(shared_cnt.at[pl.ds(el0, _CNT_PT)],
                    cnts_hbm.at[pl.ds(c * _NCNT + el0, _CNT_PT)])


@functools.cache
def _get_sc_aggregate():
    return pl.kernel(
        _sc_aggregate_body,
        out_type=(jax.ShapeDtypeStruct((_NC, _NSH, _D), jnp.float32),
                  jax.ShapeDtypeStruct((_NC * _NCNT,), jnp.float32)),
        mesh=plsc.VectorSubcoreMesh(core_axis_name="c", subcore_axis_name="s"),
        scratch_types=(
            pltpu.VMEM((_IBLK, _K), jnp.int32),
            pltpu.VMEM((_IBLK, _K), jnp.int32),
            pltpu.VMEM((2, _K, _D), jnp.float32),
            pltpu.VMEM((_K,), jnp.float32),
            pltpu.VMEM((_CNT_PT,), jnp.float32),
            pltpu.VMEM_SHARED((_NSH, _D), jnp.float32),
            pltpu.VMEM_SHARED((_NCNT,), jnp.float32),
            pltpu.SemaphoreType.DMA,
            pltpu.SemaphoreType.DMA,
        ),
    )


def _tc_dense_body(x_ref, sums_ref, cnts_ref, wl_ref, bl_ref, wr_ref, w12_ref,
                   x1_ref, out1_ref, out2_ref):
    ssum = sums_ref[0] + sums_ref[1]
    cnt = cnts_ref[0] + cnts_ref[1]
    mean = ssum / jnp.maximum(cnt, 1.0)
    x1 = (jnp.dot(mean, wl_ref[...], preferred_element_type=jnp.float32)
          + bl_ref[...]
          + jnp.dot(x_ref[...], wr_ref[...], preferred_element_type=jnp.float32))
    x1_ref[...] = x1
    rn = jnp.sqrt(jnp.sum(x1 * x1, axis=1, keepdims=True))
    hn = x1 / jnp.maximum(rn, 1e-12)
    w = w12_ref[...]
    wn = jnp.sqrt(jnp.sum(w * w, axis=0, keepdims=True))
    out12 = jnp.dot(hn, w / jnp.maximum(wn, 1e-12),
                    preferred_element_type=jnp.float32)
    out1_ref[...] = out12[:, :_C1]
    out2_ref[...] = out12[:, _D:_D + _C2]


_BR = 1000


def _tc_dense(x, sums, cnts3, w_l, b_l2, w_r, w12):
    grid = (_N // _BR,)
    return pl.pallas_call(
        _tc_dense_body,
        grid=grid,
        in_specs=[
            pl.BlockSpec((_BR, _D), lambda i: (i, 0)),
            pl.BlockSpec((_NC, _BR, _D), lambda i: (0, i, 0)),
            pl.BlockSpec((_NC, _BR, 1), lambda i: (0, i, 0)),
            pl.BlockSpec((_D, _D), lambda i: (0, 0)),
            pl.BlockSpec((1, _D), lambda i: (0, 0)),
            pl.BlockSpec((_D, _D), lambda i: (0, 0)),
            pl.BlockSpec((_D, 256), lambda i: (0, 0)),
        ],
        out_specs=[
            pl.BlockSpec((_BR, _D), lambda i: (i, 0)),
            pl.BlockSpec((_BR, _C1), lambda i: (i, 0)),
            pl.BlockSpec((_BR, _C2), lambda i: (i, 0)),
        ],
        out_shape=[
            jax.ShapeDtypeStruct((_N, _D), jnp.float32),
            jax.ShapeDtypeStruct((_N, _C1), jnp.float32),
            jax.ShapeDtypeStruct((_N, _C2), jnp.float32),
        ],
    )(x, sums, cnts3, w_l, b_l2, w_r, w12)


def kernel(x, edge_index, W_l, b_l, W_r, W1, W2):
    src = edge_index[0]
    dst = edge_index[1]
    npad = _EPAD - _E
    pad_src = (jnp.arange(npad, dtype=jnp.int32) * 97) % _N
    pad_dst = _N + (jnp.arange(npad, dtype=jnp.int32) % (_NSH - _N))
    src3 = jnp.concatenate([src, pad_src]).reshape(_NW, _CPW, _K)
    dst3 = jnp.concatenate([dst, pad_dst]).reshape(_NW, _CPW, _K)

    sums, cnts = _get_sc_aggregate()(x, src3, dst3)

    w12 = jnp.zeros((_D, 256), jnp.float32)
    w12 = lax.dynamic_update_slice(w12, W1, (0, 0))
    w12 = lax.dynamic_update_slice(w12, W2, (0, _D))
    cnts3 = jnp.stack([cnts[:_N], cnts[_NCNT:_NCNT + _N]])[:, :, None]
    x1, out1, out2 = _tc_dense(x, sums, cnts3, W_l, b_l.reshape(1, _D), W_r,
                               w12)
    return (out1, out2, x1)


# speedup vs baseline: 11.6950x; 1.0356x over previous
import functools

import jax
import jax.numpy as jnp
from jax import lax
from jax.experimental import pallas as pl
from jax.experimental.pallas import tpu as pltpu
from jax.experimental.pallas import tpu_sc as plsc

_N = 10000
_E = 320000
_D = 128
_C1 = 50
_C2 = 100

_NC = 2
_NS = 16
_NW = _NC * _NS
_K = 96
_CPW = 108
_IBLK = 24
_EPAD = _NW * _CPW * _K
_ROWS_PT = 632
_NSH = _NS * _ROWS_PT
_CNT_PT = 1024
_NCNT = _NS * _CNT_PT


def _sc_aggregate_body(x_hbm, src_hbm, dst_hbm,
                       sums_hbm, cnts_hbm,
                       idx_s_v, idx_d_v, rows_v, ones_v, zc_v,
                       shared_sum, shared_cnt,
                       gsem0, gsem1, gsem2, ssem0, ssem1, ssem2):
    gsems = (gsem0, gsem1, gsem2)
    ssems = (ssem0, ssem1, ssem2)
    c = lax.axis_index("c")
    s = lax.axis_index("s")
    wid = c * _NS + s

    def _z(i, carry):
        for l in range(_D // 16):
            rows_v[1, i, pl.ds(l * 16, 16)] = jnp.zeros((16,), jnp.float32)
        return carry

    lax.fori_loop(0, _K, _z, 0)
    for i in range(_CNT_PT // 16):
        zc_v[pl.ds(i * 16, 16)] = jnp.zeros((16,), jnp.float32)
    r0 = s * _ROWS_PT
    nfull = _ROWS_PT // _K
    for k in range(nfull):
        pltpu.sync_copy(rows_v.at[1],
                        shared_sum.at[pl.ds(r0 + k * _K, _K), :])
    rem = _ROWS_PT - nfull * _K
    pltpu.sync_copy(rows_v.at[1, pl.ds(0, rem), :],
                    shared_sum.at[pl.ds(r0 + nfull * _K, rem), :])
    pltpu.sync_copy(zc_v, shared_cnt.at[pl.ds(s * _CNT_PT, _CNT_PT)])
    for i in range(_K // 16):
        ones_v[pl.ds(i * 16, 16)] = jnp.ones((16,), jnp.float32)
    plsc.subcore_barrier()

    def _wait_gather(buf):
        pltpu.make_async_copy(x_hbm.at[pl.ds(0, _K), :], rows_v.at[buf],
                              gsems[buf]).wait()

    def _wait_scatter(buf):
        pltpu.make_async_copy(rows_v.at[buf], shared_sum.at[pl.ds(0, _K), :],
                              ssems[buf]).wait()

    def _gather(j, buf):
        pltpu.async_copy(x_hbm.at[idx_s_v.at[j]], rows_v.at[buf], gsems[buf])

    off = 0
    while off < _CPW:
        nblk = min(_IBLK, _CPW - off)
        pltpu.sync_copy(src_hbm.at[wid, pl.ds(off, nblk)],
                        idx_s_v.at[pl.ds(0, nblk)])
        pltpu.sync_copy(dst_hbm.at[wid, pl.ds(off, nblk)],
                        idx_d_v.at[pl.ds(0, nblk)])
        _gather(0, 0)
        _gather(1, 1)

        def triple(g, carry):
            for b in range(3):
                j = 3 * g + b
                nb = (b + 2) % 3
                _wait_gather(b)
                pltpu.async_copy(rows_v.at[b], shared_sum.at[idx_d_v.at[j]],
                                 ssems[b], add=True)

                @pl.when(jnp.logical_and(j >= 1, j + 2 < nblk))
                def _():
                    _wait_scatter(nb)

                @pl.when(j + 2 < nblk)
                def _():
                    _gather(j + 2, nb)

                pltpu.sync_copy(ones_v, shared_cnt.at[idx_d_v.at[j]],
                                add=True)
            return carry

        lax.fori_loop(0, nblk // 3, triple, 0)
        _wait_scatter(0)
        _wait_scatter(1)
        _wait_scatter(2)
        off += nblk
    plsc.subcore_barrier()

    row0 = s * _ROWS_PT
    pltpu.sync_copy(shared_sum.at[pl.ds(row0, _ROWS_PT), :],
                    sums_hbm.at[c, pl.ds(row0, _ROWS_PT), :])
    el0 = s * _CNT_PT
    pltpu.sync_copy(shared_cnt.at[pl.ds(el0, _CNT_PT)],
                    cnts_hbm.at[pl.ds(c * _NCNT + el0, _CNT_PT)])


@functools.cache
def _get_sc_aggregate():
    return pl.kernel(
        _sc_aggregate_body,
        out_type=(jax.ShapeDtypeStruct((_NC, _NSH, _D), jnp.float32),
                  jax.ShapeDtypeStruct((_NC * _NCNT,), jnp.float32)),
        mesh=plsc.VectorSubcoreMesh(core_axis_name="c", subcore_axis_name="s"),
        scratch_types=(
            pltpu.VMEM((_IBLK, _K), jnp.int32),
            pltpu.VMEM((_IBLK, _K), jnp.int32),
            pltpu.VMEM((3, _K, _D), jnp.float32),
            pltpu.VMEM((_K,), jnp.float32),
            pltpu.VMEM((_CNT_PT,), jnp.float32),
            pltpu.VMEM_SHARED((_NSH, _D), jnp.float32),
            pltpu.VMEM_SHARED((_NCNT,), jnp.float32),
            pltpu.SemaphoreType.DMA,
            pltpu.SemaphoreType.DMA,
            pltpu.SemaphoreType.DMA,
            pltpu.SemaphoreType.DMA,
            pltpu.SemaphoreType.DMA,
            pltpu.SemaphoreType.DMA,
        ),
    )


def _tc_dense_body(x_ref, sums_ref, cnts_ref, wl_ref, bl_ref, wr_ref, w12_ref,
                   x1_ref, out1_ref, out2_ref):
    ssum = sums_ref[0] + sums_ref[1]
    cnt = cnts_ref[0] + cnts_ref[1]
    mean = ssum / jnp.maximum(cnt, 1.0)
    x1 = (jnp.dot(mean, wl_ref[...], preferred_element_type=jnp.float32)
          + bl_ref[...]
          + jnp.dot(x_ref[...], wr_ref[...], preferred_element_type=jnp.float32))
    x1_ref[...] = x1
    rn = jnp.sqrt(jnp.sum(x1 * x1, axis=1, keepdims=True))
    hn = x1 / jnp.maximum(rn, 1e-12)
    w = w12_ref[...]
    wn = jnp.sqrt(jnp.sum(w * w, axis=0, keepdims=True))
    out12 = jnp.dot(hn, w / jnp.maximum(wn, 1e-12),
                    preferred_element_type=jnp.float32)
    out1_ref[...] = out12[:, :_C1]
    out2_ref[...] = out12[:, _D:_D + _C2]


_BR = 1000


def _tc_dense(x, sums, cnts3, w_l, b_l2, w_r, w12):
    grid = (_N // _BR,)
    return pl.pallas_call(
        _tc_dense_body,
        grid=grid,
        in_specs=[
            pl.BlockSpec((_BR, _D), lambda i: (i, 0)),
            pl.BlockSpec((_NC, _BR, _D), lambda i: (0, i, 0)),
            pl.BlockSpec((_NC, _BR, 1), lambda i: (0, i, 0)),
            pl.BlockSpec((_D, _D), lambda i: (0, 0)),
            pl.BlockSpec((1, _D), lambda i: (0, 0)),
            pl.BlockSpec((_D, _D), lambda i: (0, 0)),
            pl.BlockSpec((_D, 256), lambda i: (0, 0)),
        ],
        out_specs=[
            pl.BlockSpec((_BR, _D), lambda i: (i, 0)),
            pl.BlockSpec((_BR, _C1), lambda i: (i, 0)),
            pl.BlockSpec((_BR, _C2), lambda i: (i, 0)),
        ],
        out_shape=[
            jax.ShapeDtypeStruct((_N, _D), jnp.float32),
            jax.ShapeDtypeStruct((_N, _C1), jnp.float32),
            jax.ShapeDtypeStruct((_N, _C2), jnp.float32),
        ],
    )(x, sums, cnts3, w_l, b_l2, w_r, w12)


def kernel(x, edge_index, W_l, b_l, W_r, W1, W2):
    src = edge_index[0]
    dst = edge_index[1]
    npad = _EPAD - _E
    pad_src = (jnp.arange(npad, dtype=jnp.int32) * 97) % _N
    pad_dst = _N + (jnp.arange(npad, dtype=jnp.int32) % (_NSH - _N))
    src3 = jnp.concatenate([src, pad_src]).reshape(_NW, _CPW, _K)
    dst3 = jnp.concatenate([dst, pad_dst]).reshape(_NW, _CPW, _K)

    sums, cnts = _get_sc_aggregate()(x, src3, dst3)

    w12 = jnp.zeros((_D, 256), jnp.float32)
    w12 = lax.dynamic_update_slice(w12, W1, (0, 0))
    w12 = lax.dynamic_update_slice(w12, W2, (0, _D))
    cnts3 = jnp.stack([cnts[:_N], cnts[_NCNT:_NCNT + _N]])[:, :, None]
    x1, out1, out2 = _tc_dense(x, sums, cnts3, W_l, b_l.reshape(1, _D), W_r,
                               w12)
    return (out1, out2, x1)
